# baseline (speedup 1.0000x reference)
import math

import jax
import jax.numpy as jnp
import numpy as np
from jax.experimental import pallas as pl
from jax.experimental.pallas import tpu as pltpu

_EDGES = [float(np.float32(2.0 * math.pi) / np.float32(10.0) * np.float32(k))
          for k in range(1, 10)]
_COS = [float(np.float32(math.cos(e))) for e in _EDGES]
_SIN = [float(np.float32(math.sin(e))) for e in _EDGES]
_UPPER = [e > math.pi for e in _EDGES]

_LANES = 128
_BLOCK_ROWS = 4096


def _wedge_counts(xv, yv, acc_ref):
    y_neg = yv < 0.0
    for k in range(4):
        cross = _COS[k] * yv - _SIN[k] * xv
        pred_lo = jnp.logical_or(y_neg, cross > 0.0)
        pred_hi = jnp.logical_and(y_neg, cross < 0.0)
        acc_ref[k] += jnp.sum(jnp.where(pred_lo, 1.0, 0.0))
        acc_ref[k + 5] += jnp.sum(jnp.where(pred_hi, 1.0, 0.0))
    acc_ref[4] += jnp.sum(jnp.where(y_neg, 1.0, 0.0))


def _chi_square(nrows_total, o_ref, acc_ref):
    n_total = float(nrows_total)
    expected = float(nrows_total // 10)
    denom = expected + 1e-6
    t = [acc_ref[k] for k in range(9)]
    counts = [n_total - t[0]]
    counts += [t[j] - t[j + 1] for j in range(8)]
    counts += [t[8]]
    chi = (counts[0] - expected) ** 2 / denom
    for c in counts[1:]:
        chi = chi + (c - expected) ** 2 / denom
    o_ref[0] = chi


def kernel(x):
    n, _ = x.shape
    rows = (2 * n) // _LANES
    block_rows = min(_BLOCK_ROWS, rows)
    grid = rows // block_rows

    x0q = x[:, 0].reshape(rows, _LANES // 2)
    x1q = x[:, 1].reshape(rows, _LANES // 2)
    z = jnp.stack([x0q, x1q], axis=-1).reshape(rows, _LANES)

    def body(z_ref, o_ref, acc_ref):
        i = pl.program_id(0)

        @pl.when(i == 0)
        def _init():
            for k in range(9):
                acc_ref[k] = 0.0

        b = z_ref[...]
        half = block_rows // 2
        c0 = b[:half, :]
        c1 = b[half:, :]
        lane = jax.lax.broadcasted_iota(jnp.int32, (half, _LANES), 1)
        even = (lane % 2) == 0
        xv = jnp.where(even, c0, pltpu.roll(c1, 127, 1))
        yv = jnp.where(even, pltpu.roll(c0, 127, 1), pltpu.roll(c1, 126, 1))
        _wedge_counts(xv, yv, acc_ref)

        @pl.when(i == grid - 1)
        def _fin():
            _chi_square(n, o_ref, acc_ref)

    out = pl.pallas_call(
        body,
        grid=(grid,),
        in_specs=[pl.BlockSpec((block_rows, _LANES), lambda i: (i, 0))],
        out_specs=pl.BlockSpec(memory_space=pltpu.SMEM),
        out_shape=jax.ShapeDtypeStruct((1,), jnp.float32),
        scratch_shapes=[pltpu.SMEM((16,), jnp.float32)],
    )(z)
    return out[0]


# speedup vs baseline: 1.5019x; 1.5019x over previous
import math

import jax
import jax.numpy as jnp
import numpy as np
from jax.experimental import pallas as pl
from jax.experimental.pallas import tpu as pltpu

_EDGES = [float(np.float32(2.0 * math.pi) / np.float32(10.0) * np.float32(k))
          for k in range(1, 10)]
_COS = [float(np.float32(math.cos(e))) for e in _EDGES]
_SIN = [float(np.float32(math.sin(e))) for e in _EDGES]
_UPPER = [e > math.pi for e in _EDGES]

_LANES = 128
_BLOCK_ROWS = 4096


def _wedge_counts(xv, yv, acc_ref):
    y_neg = yv < 0.0
    for k in range(4):
        cross = _COS[k] * yv - _SIN[k] * xv
        pred_lo = jnp.logical_or(y_neg, cross > 0.0)
        pred_hi = jnp.logical_and(y_neg, cross < 0.0)
        acc_ref[k] += jnp.sum(jnp.where(pred_lo, 1.0, 0.0))
        acc_ref[k + 5] += jnp.sum(jnp.where(pred_hi, 1.0, 0.0))
    acc_ref[4] += jnp.sum(jnp.where(y_neg, 1.0, 0.0))


def _chi_square(nrows_total, o_ref, acc_ref):
    n_total = float(nrows_total)
    expected = float(nrows_total // 10)
    denom = expected + 1e-6
    t = [acc_ref[k] for k in range(9)]
    counts = [n_total - t[0]]
    counts += [t[j] - t[j + 1] for j in range(8)]
    counts += [t[8]]
    chi = (counts[0] - expected) ** 2 / denom
    for c in counts[1:]:
        chi = chi + (c - expected) ** 2 / denom
    o_ref[0] = chi


def kernel(x):
    n, _ = x.shape
    rows = (2 * n) // _LANES
    block_rows = min(_BLOCK_ROWS, rows)
    grid = rows // block_rows

    x0q = x[:, 0].reshape(rows, _LANES // 2)
    x1q = x[:, 1].reshape(rows, _LANES // 2)
    z = jnp.concatenate([x0q, x1q], axis=1)

    def body(z_ref, o_ref, acc_ref):
        i = pl.program_id(0)

        @pl.when(i == 0)
        def _init():
            for k in range(9):
                acc_ref[k] = 0.0

        b = z_ref[...]
        half = block_rows // 2
        c0 = b[:half, :]
        c1 = b[half:, :]
        lane = jax.lax.broadcasted_iota(jnp.int32, (half, _LANES), 1)
        lo = lane < (_LANES // 2)
        xv = jnp.where(lo, c0, pltpu.roll(c1, _LANES // 2, 1))
        yv = jnp.where(lo, pltpu.roll(c0, _LANES // 2, 1), c1)
        _wedge_counts(xv, yv, acc_ref)

        @pl.when(i == grid - 1)
        def _fin():
            _chi_square(n, o_ref, acc_ref)

    out = pl.pallas_call(
        body,
        grid=(grid,),
        in_specs=[pl.BlockSpec((block_rows, _LANES), lambda i: (i, 0))],
        out_specs=pl.BlockSpec(memory_space=pltpu.SMEM),
        out_shape=jax.ShapeDtypeStruct((1,), jnp.float32),
        scratch_shapes=[pltpu.SMEM((16,), jnp.float32)],
    )(z)
    return out[0]


# speedup vs baseline: 3.6068x; 2.4015x over previous
import math

import jax
import jax.numpy as jnp
import numpy as np
from jax.experimental import pallas as pl
from jax.experimental.pallas import tpu as pltpu

_EDGES = [float(np.float32(2.0 * math.pi) / np.float32(10.0) * np.float32(k))
          for k in range(1, 10)]
_COS = [float(np.float32(math.cos(e))) for e in _EDGES]
_SIN = [float(np.float32(math.sin(e))) for e in _EDGES]
_UPPER = [e > math.pi for e in _EDGES]

_LANES = 128
_BLOCK_ROWS = 4096


def _wedge_counts(xv, yv, acc_ref):
    y_neg = yv < 0.0
    for k in range(4):
        cross = _COS[k] * yv - _SIN[k] * xv
        pred_lo = jnp.logical_or(y_neg, cross > 0.0)
        pred_hi = jnp.logical_and(y_neg, cross < 0.0)
        acc_ref[k] += jnp.sum(jnp.where(pred_lo, 1.0, 0.0))
        acc_ref[k + 5] += jnp.sum(jnp.where(pred_hi, 1.0, 0.0))
    acc_ref[4] += jnp.sum(jnp.where(y_neg, 1.0, 0.0))


def _chi_square(nrows_total, o_ref, acc_ref):
    n_total = float(nrows_total)
    expected = float(nrows_total // 10)
    denom = expected + 1e-6
    t = [acc_ref[k] for k in range(9)]
    counts = [n_total - t[0]]
    counts += [t[j] - t[j + 1] for j in range(8)]
    counts += [t[8]]
    chi = (counts[0] - expected) ** 2 / denom
    for c in counts[1:]:
        chi = chi + (c - expected) ** 2 / denom
    o_ref[0] = chi


def kernel(x):
    n, _ = x.shape
    rows = n // _LANES
    block_rows = min(_BLOCK_ROWS, rows)
    grid = rows // block_rows

    x0 = x[:, 0].reshape(rows, _LANES)
    x1 = x[:, 1].reshape(rows, _LANES)

    def body(x0_ref, x1_ref, o_ref, acc_ref):
        i = pl.program_id(0)

        @pl.when(i == 0)
        def _init():
            for k in range(9):
                acc_ref[k] = 0.0

        _wedge_counts(x0_ref[...], x1_ref[...], acc_ref)

        @pl.when(i == grid - 1)
        def _fin():
            _chi_square(n, o_ref, acc_ref)

    out = pl.pallas_call(
        body,
        grid=(grid,),
        in_specs=[
            pl.BlockSpec((block_rows, _LANES), lambda i: (i, 0)),
            pl.BlockSpec((block_rows, _LANES), lambda i: (i, 0)),
        ],
        out_specs=pl.BlockSpec(memory_space=pltpu.SMEM),
        out_shape=jax.ShapeDtypeStruct((1,), jnp.float32),
        scratch_shapes=[pltpu.SMEM((16,), jnp.float32)],
    )(x0, x1)
    return out[0]


# speedup vs baseline: 3.6164x; 1.0027x over previous
import math

import jax
import jax.numpy as jnp
import numpy as np
from jax.experimental import pallas as pl
from jax.experimental.pallas import tpu as pltpu

_EDGES = [float(np.float32(2.0 * math.pi) / np.float32(10.0) * np.float32(k))
          for k in range(1, 10)]
_COS = [float(np.float32(math.cos(e))) for e in _EDGES]
_SIN = [float(np.float32(math.sin(e))) for e in _EDGES]
_UPPER = [e > math.pi for e in _EDGES]

_LANES = 128
_BLOCK_ROWS = 8192


def _wedge_counts(xv, yv, acc_ref):
    y_neg = yv < 0.0
    for k in range(4):
        cross = _COS[k] * yv - _SIN[k] * xv
        pred_lo = jnp.logical_or(y_neg, cross > 0.0)
        pred_hi = jnp.logical_and(y_neg, cross < 0.0)
        acc_ref[k] += jnp.sum(jnp.where(pred_lo, 1.0, 0.0))
        acc_ref[k + 5] += jnp.sum(jnp.where(pred_hi, 1.0, 0.0))
    acc_ref[4] += jnp.sum(jnp.where(y_neg, 1.0, 0.0))


def _chi_square(nrows_total, o_ref, acc_ref):
    n_total = float(nrows_total)
    expected = float(nrows_total // 10)
    denom = expected + 1e-6
    t = [acc_ref[k] for k in range(9)]
    counts = [n_total - t[0]]
    counts += [t[j] - t[j + 1] for j in range(8)]
    counts += [t[8]]
    chi = (counts[0] - expected) ** 2 / denom
    for c in counts[1:]:
        chi = chi + (c - expected) ** 2 / denom
    o_ref[0] = chi


def kernel(x):
    n, _ = x.shape
    rows = n // _LANES
    block_rows = min(_BLOCK_ROWS, rows)
    grid = rows // block_rows

    x0 = x[:, 0].reshape(rows, _LANES)
    x1 = x[:, 1].reshape(rows, _LANES)

    def body(x0_ref, x1_ref, o_ref, acc_ref):
        i = pl.program_id(0)

        @pl.when(i == 0)
        def _init():
            for k in range(9):
                acc_ref[k] = 0.0

        _wedge_counts(x0_ref[...], x1_ref[...], acc_ref)

        @pl.when(i == grid - 1)
        def _fin():
            _chi_square(n, o_ref, acc_ref)

    out = pl.pallas_call(
        body,
        grid=(grid,),
        in_specs=[
            pl.BlockSpec((block_rows, _LANES), lambda i: (i, 0)),
            pl.BlockSpec((block_rows, _LANES), lambda i: (i, 0)),
        ],
        out_specs=pl.BlockSpec(memory_space=pltpu.SMEM),
        out_shape=jax.ShapeDtypeStruct((1,), jnp.float32),
        scratch_shapes=[pltpu.SMEM((16,), jnp.float32)],
    )(x0, x1)
    return out[0]
